# baseline (speedup 1.0000x reference)
import functools

import jax
import jax.numpy as jnp
from jax import lax
from jax.experimental import pallas as pl
from jax.experimental.pallas import tpu as pltpu
from jax.experimental.pallas import tpu_sc as plsc

_B = 4
_F = 128
_STEP = 2
_OUTF = _F // _STEP
_VROW = 3 * 112 * 112
_AROW = 1024
_NROWS = _B * _OUTF
_NC, _NS = 2, 16
_NW = _NC * _NS
_RPW = _NROWS // _NW


def _make_sc_copy():
    mesh = plsc.VectorSubcoreMesh(
        core_axis_name="c", subcore_axis_name="s",
        num_cores=_NC, num_subcores=_NS)

    @functools.partial(
        pl.kernel,
        out_type=(
            jax.ShapeDtypeStruct((_NROWS, _VROW), jnp.float32),
            jax.ShapeDtypeStruct((_NROWS, _AROW), jnp.float32),
        ),
        mesh=mesh,
        scratch_types=[pltpu.SemaphoreType.DMA],
    )
    def sc_copy(vin, ain, vout, aout, sem):
        wid = lax.axis_index("s") * _NC + lax.axis_index("c")
        base = wid * _RPW
        copies = []
        for j in range(_RPW):
            r = base + j
            copies.append(pltpu.make_async_copy(vin.at[_STEP * r], vout.at[r], sem))
            copies.append(pltpu.make_async_copy(ain.at[_STEP * r], aout.at[r], sem))
        for c in copies:
            c.start()
        for c in copies:
            c.wait()

    return sc_copy


_sc_copy = _make_sc_copy()


def kernel(video, audio):
    vin = video.reshape(_B * _F, _VROW)
    ain = audio.reshape(_B * _F, _AROW)
    vout, aout = _sc_copy(vin, ain)
    return (vout.reshape(_B, _OUTF, 3, 112, 112),
            aout.reshape(_B, _OUTF, _AROW))
